# baseline (speedup 1.0000x reference)
import functools
import jax
import jax.numpy as jnp
from jax import lax
from jax.experimental import pallas as pl
from jax.experimental.pallas import tpu as pltpu
from jax.experimental.pallas import tpu_sc as plsc

B, N, M, C1, C2, CO = 4, 8192, 1024, 32, 64, 128
NB = 512
PTS = B * N
NW = 32
PPW = PTS // NW
CH = 128


def _nn_body(known_ref, unknown_t_ref, kf_ref, w_ref, fidx_ref, wts_ref, table_ref):
    bb = pl.program_id(0)
    kx = known_ref[:, 0:1]
    ky = known_ref[:, 1:2]
    kz = known_ref[:, 2:3]
    ux = unknown_t_ref[0:1, :]
    uy = unknown_t_ref[1:2, :]
    uz = unknown_t_ref[2:3, :]
    dx = ux - kx
    dy = uy - ky
    dz = uz - kz
    d2 = dx * dx + dy * dy + dz * dz

    sub_iota = jax.lax.broadcasted_iota(jnp.int32, (M, NB), 0)
    big = jnp.float32(jnp.inf)

    m1 = jnp.min(d2, axis=0, keepdims=True)
    i1 = jnp.min(jnp.where(d2 == m1, sub_iota, M), axis=0, keepdims=True)
    d2b = jnp.where(sub_iota == i1, big, d2)
    m2 = jnp.min(d2b, axis=0, keepdims=True)
    i2 = jnp.min(jnp.where(d2b == m2, sub_iota, M), axis=0, keepdims=True)
    d2c = jnp.where(sub_iota == i2, big, d2b)
    m3 = jnp.min(d2c, axis=0, keepdims=True)
    i3 = jnp.min(jnp.where(d2c == m3, sub_iota, M), axis=0, keepdims=True)

    def recip(m):
        return 1.0 / (jnp.sqrt(jnp.maximum(m, 0.0)) + 1e-8)

    r1, r2, r3 = recip(m1), recip(m2), recip(m3)
    norm = r1 + r2 + r3

    off = bb * M
    fidx_ref[0:1, :] = i1 + off
    fidx_ref[1:2, :] = i2 + off
    fidx_ref[2:3, :] = i3 + off
    wts_ref[0:1, :] = r1 / norm
    wts_ref[1:2, :] = r2 / norm
    wts_ref[2:3, :] = r3 / norm

    w2m = w_ref[:, 0:C2]
    table_ref[...] = jax.lax.dot_general(
        kf_ref[...], w2m, (((0,), (1,)), ((), ())),
        preferred_element_type=jnp.float32)


def _mlp_body(contrib_ref, uf_ref, w_ref, b_ref, out_ref):
    w1m = w_ref[:, C2:C2 + C1]
    out = (
        jnp.transpose(contrib_ref[...], (1, 0))
        + jnp.dot(w1m, uf_ref[...], preferred_element_type=jnp.float32)
        + b_ref[...]
    )
    out_ref[...] = jnp.maximum(out, 0.0)


def _sc_interp(table, fidx, wts):
    mesh = plsc.VectorSubcoreMesh(core_axis_name="c", subcore_axis_name="s")

    @functools.partial(
        pl.kernel,
        mesh=mesh,
        out_type=jax.ShapeDtypeStruct((PTS, CO), jnp.float32),
        scratch_types=[
            pltpu.VMEM((3, CH), jnp.int32),
            pltpu.VMEM((3, CH), jnp.float32),
            pltpu.VMEM((CH, CO), jnp.float32),
            pltpu.VMEM((CH, CO), jnp.float32),
            pltpu.VMEM((CH, CO), jnp.float32),
            pltpu.VMEM((CH, CO), jnp.float32),
            pltpu.SemaphoreType.DMA,
        ],
    )
    def run(table_hbm, fidx_hbm, wts_hbm, out_hbm,
            idx_v, w_v, r1_v, r2_v, r3_v, acc_v, sem):
        wid = lax.axis_index("c") * 16 + lax.axis_index("s")
        wbase = wid * PPW

        def chunk_body(c, _):
            base = wbase + c * CH
            pltpu.sync_copy(fidx_hbm.at[:, pl.ds(base, CH)], idx_v)
            pltpu.sync_copy(wts_hbm.at[:, pl.ds(base, CH)], w_v)
            g1 = pltpu.async_copy(table_hbm.at[idx_v.at[0]], r1_v, sem)
            g2 = pltpu.async_copy(table_hbm.at[idx_v.at[1]], r2_v, sem)
            g3 = pltpu.async_copy(table_hbm.at[idx_v.at[2]], r3_v, sem)
            g1.wait()
            g2.wait()
            g3.wait()

            def grp_body(g, _):
                w1g = w_v[0, pl.ds(g * 16, 16)]
                w2g = w_v[1, pl.ds(g * 16, 16)]
                w3g = w_v[2, pl.ds(g * 16, 16)]
                for j in range(16):
                    p = g * 16 + j
                    w1 = w1g[j]
                    w2 = w2g[j]
                    w3 = w3g[j]
                    for v in range(CO // 16):
                        sl = pl.ds(v * 16, 16)
                        acc_v[p, sl] = (r1_v[p, sl] * w1 + r2_v[p, sl] * w2
                                        + r3_v[p, sl] * w3)
                return 0

            lax.fori_loop(0, CH // 16, grp_body, 0)
            pltpu.sync_copy(acc_v, out_hbm.at[pl.ds(base, CH)])
            return 0

        lax.fori_loop(0, PPW // CH, chunk_body, 0)

    return run(table, fidx, wts)


def kernel(unknown, known, unknow_feats, known_feats, W, b):
    unknown_t = jnp.transpose(unknown, (0, 2, 1))
    b2 = b.reshape(CO, 1)

    grid = (B, N // NB)
    fidx, wts, table = pl.pallas_call(
        _nn_body,
        grid=grid,
        in_specs=[
            pl.BlockSpec((None, M, 3), lambda bb, nn: (bb, 0, 0)),
            pl.BlockSpec((None, 3, NB), lambda bb, nn: (bb, 0, nn)),
            pl.BlockSpec((None, C2, M), lambda bb, nn: (bb, 0, 0)),
            pl.BlockSpec((CO, C1 + C2), lambda bb, nn: (0, 0)),
        ],
        out_specs=[
            pl.BlockSpec((3, NB), lambda bb, nn: (0, bb * (N // NB) + nn)),
            pl.BlockSpec((3, NB), lambda bb, nn: (0, bb * (N // NB) + nn)),
            pl.BlockSpec((M, CO), lambda bb, nn: (bb, 0)),
        ],
        out_shape=[
            jax.ShapeDtypeStruct((3, PTS), jnp.int32),
            jax.ShapeDtypeStruct((3, PTS), jnp.float32),
            jax.ShapeDtypeStruct((B * M, CO), jnp.float32),
        ],
        compiler_params=pltpu.CompilerParams(
            dimension_semantics=("parallel", "arbitrary"),
        ),
    )(known, unknown_t, known_feats, W)

    contrib = _sc_interp(table, fidx, wts).reshape(B, N, CO)

    out = pl.pallas_call(
        _mlp_body,
        grid=grid,
        in_specs=[
            pl.BlockSpec((None, NB, CO), lambda bb, nn: (bb, nn, 0)),
            pl.BlockSpec((None, C1, NB), lambda bb, nn: (bb, 0, nn)),
            pl.BlockSpec((CO, C1 + C2), lambda bb, nn: (0, 0)),
            pl.BlockSpec((CO, 1), lambda bb, nn: (0, 0)),
        ],
        out_specs=pl.BlockSpec((None, CO, NB), lambda bb, nn: (bb, 0, nn)),
        out_shape=jax.ShapeDtypeStruct((B, CO, N), jnp.float32),
        compiler_params=pltpu.CompilerParams(
            dimension_semantics=("parallel", "parallel"),
        ),
    )(contrib, unknow_feats, W, b2)
    return out


# speedup vs baseline: 1.0567x; 1.0567x over previous
import functools
import jax
import jax.numpy as jnp
from jax import lax
from jax.experimental import pallas as pl
from jax.experimental.pallas import tpu as pltpu
from jax.experimental.pallas import tpu_sc as plsc

B, N, M, C1, C2, CO = 4, 8192, 1024, 32, 64, 128
NB = 1024
PTS = B * N
NW = 32
PPW = PTS // NW
CH = 128


def _nn_body(known_ref, unknown_t_ref, kf_ref, w_ref, fidx_ref, wts_ref, table_ref):
    bb = pl.program_id(0)
    kx = known_ref[:, 0:1]
    ky = known_ref[:, 1:2]
    kz = known_ref[:, 2:3]
    ux = unknown_t_ref[0:1, :]
    uy = unknown_t_ref[1:2, :]
    uz = unknown_t_ref[2:3, :]
    dx = ux - kx
    dy = uy - ky
    dz = uz - kz
    d2 = dx * dx + dy * dy + dz * dz

    sub_iota = jax.lax.broadcasted_iota(jnp.int32, (M, NB), 0)
    big = jnp.float32(jnp.inf)

    m1 = jnp.min(d2, axis=0, keepdims=True)
    i1 = jnp.min(jnp.where(d2 == m1, sub_iota, M), axis=0, keepdims=True)
    d2b = jnp.where(sub_iota == i1, big, d2)
    m2 = jnp.min(d2b, axis=0, keepdims=True)
    i2 = jnp.min(jnp.where(d2b == m2, sub_iota, M), axis=0, keepdims=True)
    d2c = jnp.where(sub_iota == i2, big, d2b)
    m3 = jnp.min(d2c, axis=0, keepdims=True)
    i3 = jnp.min(jnp.where(d2c == m3, sub_iota, M), axis=0, keepdims=True)

    def recip(m):
        return 1.0 / (jnp.sqrt(jnp.maximum(m, 0.0)) + 1e-8)

    r1, r2, r3 = recip(m1), recip(m2), recip(m3)
    norm = r1 + r2 + r3

    off = bb * M
    fidx_ref[0:1, :] = i1 + off
    fidx_ref[1:2, :] = i2 + off
    fidx_ref[2:3, :] = i3 + off
    wts_ref[0:1, :] = r1 / norm
    wts_ref[1:2, :] = r2 / norm
    wts_ref[2:3, :] = r3 / norm

    w2m = w_ref[:, 0:C2]
    table_ref[...] = jax.lax.dot_general(
        kf_ref[...], w2m, (((0,), (1,)), ((), ())),
        preferred_element_type=jnp.float32)


def _mlp_body(contrib_ref, uf_ref, w_ref, b_ref, out_ref):
    w1m = w_ref[:, C2:C2 + C1]
    out = (
        jnp.transpose(contrib_ref[...], (1, 0))
        + jnp.dot(w1m, uf_ref[...], preferred_element_type=jnp.float32)
        + b_ref[...]
    )
    out_ref[...] = jnp.maximum(out, 0.0)


def _sc_interp(table, fidx, wts):
    mesh = plsc.VectorSubcoreMesh(core_axis_name="c", subcore_axis_name="s")
    NCH = PPW // CH

    @functools.partial(
        pl.kernel,
        mesh=mesh,
        out_type=jax.ShapeDtypeStruct((PTS, CO), jnp.float32),
        scratch_types=[
            pltpu.VMEM((2, 3, CH), jnp.int32),
            pltpu.VMEM((2, 3, CH), jnp.float32),
            pltpu.VMEM((2, CH, CO), jnp.float32),
            pltpu.VMEM((2, CH, CO), jnp.float32),
            pltpu.VMEM((2, CH, CO), jnp.float32),
            pltpu.VMEM((CH, CO), jnp.float32),
            pltpu.SemaphoreType.DMA,
            pltpu.SemaphoreType.DMA,
        ],
    )
    def run(table_hbm, fidx_hbm, wts_hbm, out_hbm,
            idx_v, w_v, r1_v, r2_v, r3_v, acc_v, gsem, wsem):
        wid = lax.axis_index("c") * 16 + lax.axis_index("s")
        wbase = wid * PPW

        def fire(c, s):
            base = wbase + c * CH
            pltpu.sync_copy(fidx_hbm.at[:, pl.ds(base, CH)], idx_v.at[s])
            pltpu.sync_copy(wts_hbm.at[:, pl.ds(base, CH)], w_v.at[s])
            pltpu.async_copy(table_hbm.at[idx_v.at[s, 0]], r1_v.at[s], gsem)
            pltpu.async_copy(table_hbm.at[idx_v.at[s, 1]], r2_v.at[s], gsem)
            pltpu.async_copy(table_hbm.at[idx_v.at[s, 2]], r3_v.at[s], gsem)

        def drain(s):
            pltpu.make_async_copy(table_hbm.at[idx_v.at[s, 0]], r1_v.at[s], gsem).wait()
            pltpu.make_async_copy(table_hbm.at[idx_v.at[s, 1]], r2_v.at[s], gsem).wait()
            pltpu.make_async_copy(table_hbm.at[idx_v.at[s, 2]], r3_v.at[s], gsem).wait()

        fire(0, 0)

        def chunk_body(c, _):
            s = lax.rem(c, 2)
            sn = lax.rem(c + 1, 2)

            @pl.when(c + 1 < NCH)
            def _():
                fire(c + 1, sn)

            drain(s)

            def grp_body(g, _):
                w1g = w_v[s, 0, pl.ds(g * 16, 16)]
                w2g = w_v[s, 1, pl.ds(g * 16, 16)]
                w3g = w_v[s, 2, pl.ds(g * 16, 16)]
                for j in range(16):
                    p = g * 16 + j
                    w1 = w1g[j]
                    w2 = w2g[j]
                    w3 = w3g[j]
                    for v in range(CO // 16):
                        sl = pl.ds(v * 16, 16)
                        acc_v[p, sl] = (r1_v[s, p, sl] * w1 + r2_v[s, p, sl] * w2
                                        + r3_v[s, p, sl] * w3)
                return 0

            lax.fori_loop(0, CH // 16, grp_body, 0)
            pltpu.sync_copy(acc_v, out_hbm.at[pl.ds(wbase + c * CH, CH)])
            return 0

        lax.fori_loop(0, NCH, chunk_body, 0)

    return run(table, fidx, wts)


def kernel(unknown, known, unknow_feats, known_feats, W, b):
    unknown_t = jnp.transpose(unknown, (0, 2, 1))
    b2 = b.reshape(CO, 1)

    grid = (B, N // NB)
    fidx, wts, table = pl.pallas_call(
        _nn_body,
        grid=grid,
        in_specs=[
            pl.BlockSpec((None, M, 3), lambda bb, nn: (bb, 0, 0)),
            pl.BlockSpec((None, 3, NB), lambda bb, nn: (bb, 0, nn)),
            pl.BlockSpec((None, C2, M), lambda bb, nn: (bb, 0, 0)),
            pl.BlockSpec((CO, C1 + C2), lambda bb, nn: (0, 0)),
        ],
        out_specs=[
            pl.BlockSpec((3, NB), lambda bb, nn: (0, bb * (N // NB) + nn)),
            pl.BlockSpec((3, NB), lambda bb, nn: (0, bb * (N // NB) + nn)),
            pl.BlockSpec((M, CO), lambda bb, nn: (bb, 0)),
        ],
        out_shape=[
            jax.ShapeDtypeStruct((3, PTS), jnp.int32),
            jax.ShapeDtypeStruct((3, PTS), jnp.float32),
            jax.ShapeDtypeStruct((B * M, CO), jnp.float32),
        ],
        compiler_params=pltpu.CompilerParams(
            dimension_semantics=("parallel", "arbitrary"),
        ),
    )(known, unknown_t, known_feats, W)

    contrib = _sc_interp(table, fidx, wts).reshape(B, N, CO)

    out = pl.pallas_call(
        _mlp_body,
        grid=grid,
        in_specs=[
            pl.BlockSpec((None, NB, CO), lambda bb, nn: (bb, nn, 0)),
            pl.BlockSpec((None, C1, NB), lambda bb, nn: (bb, 0, nn)),
            pl.BlockSpec((CO, C1 + C2), lambda bb, nn: (0, 0)),
            pl.BlockSpec((CO, 1), lambda bb, nn: (0, 0)),
        ],
        out_specs=pl.BlockSpec((None, CO, NB), lambda bb, nn: (bb, 0, nn)),
        out_shape=jax.ShapeDtypeStruct((B, CO, N), jnp.float32),
        compiler_params=pltpu.CompilerParams(
            dimension_semantics=("parallel", "parallel"),
        ),
    )(contrib, unknow_feats, W, b2)
    return out


# speedup vs baseline: 1.2090x; 1.1441x over previous
import functools
import jax
import jax.numpy as jnp
from jax import lax
from jax.experimental import pallas as pl
from jax.experimental.pallas import tpu as pltpu
from jax.experimental.pallas import tpu_sc as plsc

B, N, M, C1, C2, CO = 4, 8192, 1024, 32, 64, 128
NH = 2
N2 = N // NH
NB = 1024
PTSH = B * N2
NW = 32
PPW = PTSH // NW
CH = 128
NCH = PPW // CH


def _nn_body(known_ref, unknown_t_ref, kf_ref, w_ref, fidx_ref, wts_ref, table_ref):
    bb = pl.program_id(0)
    kx = known_ref[:, 0:1]
    ky = known_ref[:, 1:2]
    kz = known_ref[:, 2:3]
    ux = unknown_t_ref[0:1, :]
    uy = unknown_t_ref[1:2, :]
    uz = unknown_t_ref[2:3, :]
    dx = ux - kx
    dy = uy - ky
    dz = uz - kz
    d2 = dx * dx + dy * dy + dz * dz

    sub_iota = jax.lax.broadcasted_iota(jnp.int32, (M, NB), 0)
    big = jnp.float32(jnp.inf)

    m1 = jnp.min(d2, axis=0, keepdims=True)
    i1 = jnp.min(jnp.where(d2 == m1, sub_iota, M), axis=0, keepdims=True)
    d2b = jnp.where(sub_iota == i1, big, d2)
    m2 = jnp.min(d2b, axis=0, keepdims=True)
    i2 = jnp.min(jnp.where(d2b == m2, sub_iota, M), axis=0, keepdims=True)
    d2c = jnp.where(sub_iota == i2, big, d2b)
    m3 = jnp.min(d2c, axis=0, keepdims=True)
    i3 = jnp.min(jnp.where(d2c == m3, sub_iota, M), axis=0, keepdims=True)

    def recip(m):
        return 1.0 / (jnp.sqrt(jnp.maximum(m, 0.0)) + 1e-8)

    r1, r2, r3 = recip(m1), recip(m2), recip(m3)
    norm = r1 + r2 + r3

    off = bb * M
    fidx_ref[0:1, :] = i1 + off
    fidx_ref[1:2, :] = i2 + off
    fidx_ref[2:3, :] = i3 + off
    wts_ref[0:1, :] = r1 / norm
    wts_ref[1:2, :] = r2 / norm
    wts_ref[2:3, :] = r3 / norm

    w2m = w_ref[:, 0:C2]
    table_ref[...] = jax.lax.dot_general(
        kf_ref[...], w2m, (((0,), (1,)), ((), ())),
        preferred_element_type=jnp.float32)


def _mlp_body(contrib_ref, uf_ref, w_ref, b_ref, out_ref):
    w1m = w_ref[:, C2:C2 + C1]
    out = (
        jnp.transpose(contrib_ref[...], (1, 0))
        + jnp.dot(w1m, uf_ref[...], preferred_element_type=jnp.float32)
        + b_ref[...]
    )
    out_ref[...] = jnp.maximum(out, 0.0)


def _sc_interp(table, fidx, wts):
    mesh = plsc.VectorSubcoreMesh(core_axis_name="c", subcore_axis_name="s")

    @functools.partial(
        pl.kernel,
        mesh=mesh,
        out_type=jax.ShapeDtypeStruct((PTSH, CO), jnp.float32),
        scratch_types=[
            pltpu.VMEM((2, 3, CH), jnp.int32),
            pltpu.VMEM((2, 3, CH), jnp.float32),
            pltpu.VMEM((2, CH, CO), jnp.float32),
            pltpu.VMEM((2, CH, CO), jnp.float32),
            pltpu.VMEM((2, CH, CO), jnp.float32),
            pltpu.VMEM((CH, CO), jnp.float32),
            pltpu.SemaphoreType.DMA,
            pltpu.SemaphoreType.DMA,
        ],
    )
    def run(table_hbm, fidx_hbm, wts_hbm, out_hbm,
            idx_v, w_v, r1_v, r2_v, r3_v, acc_v, gsem, wsem):
        wid = lax.axis_index("c") * 16 + lax.axis_index("s")
        wbase = wid * PPW

        def fire(c, s):
            base = wbase + c * CH
            pltpu.sync_copy(fidx_hbm.at[:, pl.ds(base, CH)], idx_v.at[s])
            pltpu.sync_copy(wts_hbm.at[:, pl.ds(base, CH)], w_v.at[s])
            pltpu.async_copy(table_hbm.at[idx_v.at[s, 0]], r1_v.at[s], gsem)
            pltpu.async_copy(table_hbm.at[idx_v.at[s, 1]], r2_v.at[s], gsem)
            pltpu.async_copy(table_hbm.at[idx_v.at[s, 2]], r3_v.at[s], gsem)

        def drain(s):
            pltpu.make_async_copy(table_hbm.at[idx_v.at[s, 0]], r1_v.at[s], gsem).wait()
            pltpu.make_async_copy(table_hbm.at[idx_v.at[s, 1]], r2_v.at[s], gsem).wait()
            pltpu.make_async_copy(table_hbm.at[idx_v.at[s, 2]], r3_v.at[s], gsem).wait()

        fire(0, 0)

        def chunk_body(c, _):
            s = lax.rem(c, 2)
            sn = lax.rem(c + 1, 2)

            @pl.when(c + 1 < NCH)
            def _():
                fire(c + 1, sn)

            drain(s)

            def grp_body(g, _):
                w1g = w_v[s, 0, pl.ds(g * 16, 16)]
                w2g = w_v[s, 1, pl.ds(g * 16, 16)]
                w3g = w_v[s, 2, pl.ds(g * 16, 16)]
                for j in range(16):
                    p = g * 16 + j
                    w1 = w1g[j]
                    w2 = w2g[j]
                    w3 = w3g[j]
                    for v in range(CO // 16):
                        sl = pl.ds(v * 16, 16)
                        acc_v[p, sl] = (r1_v[s, p, sl] * w1 + r2_v[s, p, sl] * w2
                                        + r3_v[s, p, sl] * w3)
                return 0

            lax.fori_loop(0, CH // 16, grp_body, 0)
            pltpu.sync_copy(acc_v, out_hbm.at[pl.ds(wbase + c * CH, CH)])
            return 0

        lax.fori_loop(0, NCH, chunk_body, 0)

    return run(table, fidx, wts)


def _half(known, unknown_t_h, uf_h, known_feats, W, b2):
    grid = (B, N2 // NB)
    fidx, wts, table = pl.pallas_call(
        _nn_body,
        grid=grid,
        in_specs=[
            pl.BlockSpec((None, M, 3), lambda bb, nn: (bb, 0, 0)),
            pl.BlockSpec((None, 3, NB), lambda bb, nn: (bb, 0, nn)),
            pl.BlockSpec((None, C2, M), lambda bb, nn: (bb, 0, 0)),
            pl.BlockSpec((CO, C1 + C2), lambda bb, nn: (0, 0)),
        ],
        out_specs=[
            pl.BlockSpec((3, NB), lambda bb, nn: (0, bb * (N2 // NB) + nn)),
            pl.BlockSpec((3, NB), lambda bb, nn: (0, bb * (N2 // NB) + nn)),
            pl.BlockSpec((M, CO), lambda bb, nn: (bb, 0)),
        ],
        out_shape=[
            jax.ShapeDtypeStruct((3, PTSH), jnp.int32),
            jax.ShapeDtypeStruct((3, PTSH), jnp.float32),
            jax.ShapeDtypeStruct((B * M, CO), jnp.float32),
        ],
        compiler_params=pltpu.CompilerParams(
            dimension_semantics=("parallel", "arbitrary"),
        ),
    )(known, unknown_t_h, known_feats, W)

    contrib = _sc_interp(table, fidx, wts).reshape(B, N2, CO)

    out = pl.pallas_call(
        _mlp_body,
        grid=grid,
        in_specs=[
            pl.BlockSpec((None, NB, CO), lambda bb, nn: (bb, nn, 0)),
            pl.BlockSpec((None, C1, NB), lambda bb, nn: (bb, 0, nn)),
            pl.BlockSpec((CO, C1 + C2), lambda bb, nn: (0, 0)),
            pl.BlockSpec((CO, 1), lambda bb, nn: (0, 0)),
        ],
        out_specs=pl.BlockSpec((None, CO, NB), lambda bb, nn: (bb, 0, nn)),
        out_shape=jax.ShapeDtypeStruct((B, CO, N2), jnp.float32),
        compiler_params=pltpu.CompilerParams(
            dimension_semantics=("parallel", "parallel"),
        ),
    )(contrib, uf_h, W, b2)
    return out


def kernel(unknown, known, unknow_feats, known_feats, W, b):
    unknown_t = jnp.transpose(unknown, (0, 2, 1))
    b2 = b.reshape(CO, 1)

    outs = []
    for h in range(NH):
        sl = slice(h * N2, (h + 1) * N2)
        outs.append(_half(known, unknown_t[:, :, sl], unknow_feats[:, :, sl],
                          known_feats, W, b2))
    return jnp.concatenate(outs, axis=2)
